# baseline (speedup 1.0000x reference)
import functools

import jax
import jax.numpy as jnp
from jax import lax
from jax.experimental import pallas as pl
from jax.experimental.pallas import tpu as pltpu
from jax.experimental.pallas import tpu_sc as plsc

_K = 8
_R = 16
_NEG = -3.0e38


def _score_body(k_ratio, llm_q_ref, vit_q_ref, kl_ref, kv_ref, score_ref):
    kl = kl_ref[...]
    kv = kv_ref[...]
    nl = jnp.sqrt(jnp.sum(kl * kl, axis=1, keepdims=True))
    cl = kl / jnp.maximum(nl, 1e-12)
    nv = jnp.sqrt(jnp.sum(kv * kv, axis=1, keepdims=True))
    cv = kv / jnp.maximum(nv, 1e-12)
    nt = (((1,), (1,)), ((), ()))
    s = jax.lax.dot_general(llm_q_ref[...], cl, nt,
                            precision=jax.lax.Precision.HIGHEST,
                            preferred_element_type=jnp.float32)
    s = s + k_ratio * jax.lax.dot_general(vit_q_ref[...], cv, nt,
                                          precision=jax.lax.Precision.HIGHEST,
                                          preferred_element_type=jnp.float32)
    score_ref[...] = s


def _score(llm_query, vit_query, keys_llm, keys_vit, k_ratio):
    b = llm_query.shape[0]
    e = keys_llm.shape[0]
    return pl.pallas_call(
        functools.partial(_score_body, k_ratio),
        out_shape=jax.ShapeDtypeStruct((b, e), jnp.float32),
    )(llm_query, vit_query, keys_llm, keys_vit)


def _select_body(score_hbm, idx_hbm, gatev_hbm, sc_v, idx_v, gv_v,
                 tmp_f, tmp_i):
    cid = lax.axis_index("c")
    sid = lax.axis_index("s")
    nb = score_hbm.shape[0]
    ne = score_hbm.shape[1]
    nc = ne // 16

    @pl.when((cid == 0) & (sid < nb))
    def _():
        b = sid
        pltpu.sync_copy(score_hbm.at[b], sc_v)
        iota = lax.iota(jnp.int32, 16)

        def _butterfly(v, tmp_ref, op):
            for sh in (8, 4, 2, 1):
                tmp_ref[...] = v
                v = op(v, plsc.load_gather(tmp_ref, [iota ^ sh]))
            return v

        def _maxsplat(v):
            return _butterfly(v, tmp_f, jnp.maximum)

        def _minsplat_i32(v):
            return _butterfly(v, tmp_i, jnp.minimum)

        def _sumsplat(v):
            return _butterfly(v, tmp_f, jnp.add)
        vs = [sc_v[pl.ds(c * 16, 16)] for c in range(nc)]
        idxv = jnp.zeros((16,), jnp.int32)
        vv = jnp.zeros((16,), jnp.float32)
        m0 = None
        for j in range(_K):
            v4 = vs[0]
            for c in range(1, nc):
                v4 = jnp.maximum(v4, vs[c])
            m = _maxsplat(v4)
            cand = [jnp.where(vs[c] == m, iota + c * 16, ne) for c in range(nc)]
            cmin = cand[0]
            for c in range(1, nc):
                cmin = jnp.minimum(cmin, cand[c])
            ij = _minsplat_i32(cmin)
            if j == 0:
                m0 = m
            idxv = jnp.where(iota == j, ij, idxv)
            vv = jnp.where(iota == j, m, vv)
            vs = [jnp.where(iota + c * 16 == ij, _NEG, vs[c])
                  for c in range(nc)]
        e = jnp.where(iota < _K, jnp.exp(vv - m0), 0.0)
        g = e / _sumsplat(e)
        idx_v[...] = idxv
        for k in range(_K):
            gv_v[pl.ds(k * _R, _R)] = _maxsplat(
                jnp.where(iota == k, g, -1.0))
        gv_v[pl.ds(_K * _R, _R)] = jnp.full((_R,), 1.0, jnp.float32)
        pltpu.sync_copy(idx_v, idx_hbm.at[b])
        pltpu.sync_copy(gv_v, gatev_hbm.at[b])


def _select(scores):
    b, e = scores.shape
    w = (_K + 1) * _R
    mesh = plsc.VectorSubcoreMesh(core_axis_name="c", subcore_axis_name="s",
                                  num_cores=2, num_subcores=16)
    fn = functools.partial(
        pl.kernel,
        out_type=(
            jax.ShapeDtypeStruct((b, 16), jnp.int32),
            jax.ShapeDtypeStruct((b, w), jnp.float32),
        ),
        mesh=mesh,
        scratch_types=[
            pltpu.VMEM((e,), jnp.float32),
            pltpu.VMEM((16,), jnp.int32),
            pltpu.VMEM((w,), jnp.float32),
            pltpu.VMEM((16,), jnp.float32),
            pltpu.VMEM((16,), jnp.int32),
        ],
        compiler_params=pltpu.CompilerParams(needs_layout_passes=False),
    )(_select_body)
    return fn(scores)


def _dense_body(idx_ref, x_ref, g_ref, *rest):
    a_refs = rest[:_K + 1]
    b_refs = rest[_K + 1:2 * (_K + 1)]
    out_ref = rest[2 * (_K + 1)]
    w1_s, w2_s = rest[2 * (_K + 1) + 1], rest[2 * (_K + 1) + 2]
    r = 16
    s_id = pl.program_id(1)

    @pl.when(s_id == 0)
    def _build():
        for k in range(_K + 1):
            ak = a_refs[k][...]
            bk = b_refs[k][...]
            if ak.ndim == 3:
                ak = ak[0]
                bk = bk[0]
            w1_s[:, k * r:(k + 1) * r] = ak.astype(jnp.bfloat16)
            w2_s[k * r:(k + 1) * r, :] = bk.astype(jnp.bfloat16)

    xb = x_ref[0]
    hid = jnp.dot(xb.astype(jnp.bfloat16), w1_s[...],
                  preferred_element_type=jnp.float32)
    hid = hid * g_ref[0]
    lora = jnp.dot(hid.astype(jnp.bfloat16), w2_s[...],
                   preferred_element_type=jnp.float32)
    out_ref[0] = xb + lora


def _dense(x, gate_vec, idx, a_pool, b_pool, share_a, share_b, s_blk=256):
    bsz, s, d = x.shape
    e, _, r = a_pool.shape
    w = (_K + 1) * r

    def im_x(b, sb, idx_ref):
        return (b, sb, 0)

    def im_g(b, sb, idx_ref):
        return (b, 0, 0)

    in_specs = [
        pl.BlockSpec((1, s_blk, d), im_x),
        pl.BlockSpec((1, 1, w), im_g),
    ]
    for k in range(_K):
        in_specs.append(pl.BlockSpec(
            (1, d, r), lambda b, sb, idx_ref, k=k: (idx_ref[b, k], 0, 0)))
    in_specs.append(pl.BlockSpec((d, r), lambda b, sb, idx_ref: (0, 0)))
    for k in range(_K):
        in_specs.append(pl.BlockSpec(
            (1, r, d), lambda b, sb, idx_ref, k=k: (idx_ref[b, k], 0, 0)))
    in_specs.append(pl.BlockSpec((r, d), lambda b, sb, idx_ref: (0, 0)))

    grid_spec = pltpu.PrefetchScalarGridSpec(
        num_scalar_prefetch=1,
        grid=(bsz, s // s_blk),
        in_specs=in_specs,
        out_specs=pl.BlockSpec((1, s_blk, d), im_x),
        scratch_shapes=[
            pltpu.VMEM((d, w), jnp.bfloat16),
            pltpu.VMEM((w, d), jnp.bfloat16),
        ],
    )
    args = [x, gate_vec]
    args += [a_pool] * _K + [share_a] + [b_pool] * _K + [share_b]
    return pl.pallas_call(
        _dense_body,
        grid_spec=grid_spec,
        out_shape=jax.ShapeDtypeStruct((bsz, s, d), jnp.float32),
        compiler_params=pltpu.CompilerParams(
            dimension_semantics=("arbitrary", "arbitrary"),
        ),
    )(idx, *args)


def kernel(x, llm_query, vit_query, keys_llm, keys_vit, A_pool, B_pool,
           share_A, share_B, top_k):
    del top_k
    bsz = x.shape[0]
    k_ratio = keys_vit.shape[1] / keys_llm.shape[1]
    scores = _score(llm_query, vit_query, keys_llm, keys_vit, k_ratio)
    idx, gatev = _select(scores)
    gate_vec = gatev.reshape(bsz, 1, (_K + 1) * _R)
    return _dense(x, gate_vec, idx, A_pool, B_pool, share_A, share_B)
